# baseline (speedup 1.0000x reference)
import functools

import jax
import jax.numpy as jnp
from jax import lax
from jax.experimental import pallas as pl
from jax.experimental.pallas import tpu as pltpu
from jax.experimental.pallas import tpu_sc as plsc

N_NODES = 10000
N_EDGES = 320000
FEAT = 128
NUM_GRAPHS = 64

NC = 2
NS = 16
NW = NC * NS
E_PER_W = N_EDGES // NW
CHUNK = 80
N_CHUNKS = E_PER_W // CHUNK
ROWS_A = 624
ROWS_LAST = N_NODES - (NS - 1) * ROWS_A
ZR = 8

BLK = 1000
N_BLKS = N_NODES // BLK


def _seg_sum_body(h_hbm, src_hbm, dst_hbm, out_hbm,
                  src_v, dst_v, rows0_v, rows1_v, zbuf_v, agg_sh,
                  sem0, sem1, semz):
    c = lax.axis_index("c")
    s = lax.axis_index("s")
    wid = s * NC + c

    cs = pltpu.async_copy(src_hbm.at[wid], src_v, sem0)
    cd = pltpu.async_copy(dst_hbm.at[wid], dst_v, sem1)

    zeros16 = jnp.zeros((16,), jnp.float32)

    @pl.loop(0, ZR)
    def _zero_rows(r):
        for cc in range(FEAT // 16):
            zbuf_v[r, pl.ds(cc * 16, 16)] = zeros16

    r0 = s * ROWS_A

    nz = jnp.where(s == NS - 1, ROWS_LAST // ZR, ROWS_A // ZR)

    @pl.loop(0, nz)
    def _zero_issue(j):
        pltpu.async_copy(zbuf_v, agg_sh.at[pl.ds(r0 + j * ZR, ZR)], semz)

    cs.wait()
    cd.wait()
    pltpu.async_copy(h_hbm.at[src_v.at[pl.ds(0, CHUNK)]], rows0_v, sem0)
    pltpu.async_copy(h_hbm.at[src_v.at[pl.ds(CHUNK, CHUNK)]], rows1_v, sem1)

    @pl.loop(0, nz)
    def _zero_drain(j):
        pltpu.make_async_copy(zbuf_v, agg_sh.at[pl.ds(r0 + j * ZR, ZR)],
                              semz).wait()

    plsc.subcore_barrier()

    @pl.loop(0, N_CHUNKS - 1, step=2)
    def _edges(i):
        pltpu.make_async_copy(
            h_hbm.at[src_v.at[pl.ds(i * CHUNK, CHUNK)]], rows0_v, sem0).wait()
        pltpu.sync_copy(rows0_v, agg_sh.at[dst_v.at[i]], add=True)
        pltpu.async_copy(
            h_hbm.at[src_v.at[pl.ds((i + 2) * CHUNK, CHUNK)]], rows0_v, sem0)
        pltpu.make_async_copy(
            h_hbm.at[src_v.at[pl.ds((i + 1) * CHUNK, CHUNK)]], rows1_v, sem1).wait()
        pltpu.sync_copy(rows1_v, agg_sh.at[dst_v.at[i + 1]], add=True)

        @pl.when(i + 3 < N_CHUNKS)
        def _():
            pltpu.async_copy(
                h_hbm.at[src_v.at[pl.ds((i + 3) * CHUNK, CHUNK)]], rows1_v, sem1)

    pltpu.make_async_copy(
        h_hbm.at[src_v.at[pl.ds((N_CHUNKS - 1) * CHUNK, CHUNK)]],
        rows0_v, sem0).wait()
    pltpu.sync_copy(rows0_v, agg_sh.at[dst_v.at[N_CHUNKS - 1]], add=True)

    plsc.subcore_barrier()

    @pl.when(s < NS - 1)
    def _():
        pltpu.sync_copy(agg_sh.at[pl.ds(r0, ROWS_A)],
                        out_hbm.at[c, pl.ds(r0, ROWS_A)])

    @pl.when(s == NS - 1)
    def _():
        pltpu.sync_copy(agg_sh.at[pl.ds(r0, ROWS_LAST)],
                        out_hbm.at[c, pl.ds(r0, ROWS_LAST)])


_seg_sum = functools.partial(
    pl.kernel,
    out_type=jax.ShapeDtypeStruct((NC, N_NODES, FEAT), jnp.float32),
    mesh=plsc.VectorSubcoreMesh(core_axis_name="c", subcore_axis_name="s",
                                num_cores=NC, num_subcores=NS),
    scratch_types=[
        pltpu.VMEM((E_PER_W,), jnp.int32),
        pltpu.VMEM((N_CHUNKS, CHUNK), jnp.int32),
        pltpu.VMEM((CHUNK, FEAT), jnp.float32),
        pltpu.VMEM((CHUNK, FEAT), jnp.float32),
        pltpu.VMEM((ZR, FEAT), jnp.float32),
        pltpu.VMEM_SHARED((N_NODES, FEAT), jnp.float32),
        pltpu.SemaphoreType.DMA,
        pltpu.SemaphoreType.DMA,
        pltpu.SemaphoreType.DMA,
    ],
)(_seg_sum_body)


def _mlp_body(h_ref, a0_ref, a1_ref, b2d_ref, w1_ref, b1_ref, w2_ref, b2_ref,
              hout_ref, pool_ref):
    z = h_ref[...] + a0_ref[...] + a1_ref[...]
    z = jnp.maximum(
        jnp.dot(z.astype(jnp.bfloat16), w1_ref[...].astype(jnp.bfloat16),
                preferred_element_type=jnp.float32) + b1_ref[...], 0.0)
    z = jnp.dot(z.astype(jnp.bfloat16), w2_ref[...].astype(jnp.bfloat16),
                preferred_element_type=jnp.float32) + b2_ref[...]
    h = jnp.maximum(z, 0.0)
    hout_ref[...] = h

    bids = b2d_ref[0, :]
    onehot = (bids[None, :]
              == lax.broadcasted_iota(jnp.int32, (NUM_GRAPHS, N_NODES), 0)
              ).astype(jnp.float32)
    pool_ref[...] = lax.dot_general(onehot, h, (((1,), (0,)), ((), ())),
                                    preferred_element_type=jnp.float32)


_mlp = pl.pallas_call(
    _mlp_body,
    out_shape=[
        jax.ShapeDtypeStruct((N_NODES, FEAT), jnp.float32),
        jax.ShapeDtypeStruct((NUM_GRAPHS, FEAT), jnp.float32),
    ],
)


def _mlp_last_body(h_ref, a0_ref, a1_ref, b2d_ref, w1_ref, b1_ref, w2_ref,
                   b2_ref, pool_ref):
    z = h_ref[...] + a0_ref[...] + a1_ref[...]
    z = jnp.maximum(
        jnp.dot(z.astype(jnp.bfloat16), w1_ref[...].astype(jnp.bfloat16),
                preferred_element_type=jnp.float32) + b1_ref[...], 0.0)
    z = jnp.dot(z.astype(jnp.bfloat16), w2_ref[...].astype(jnp.bfloat16),
                preferred_element_type=jnp.float32) + b2_ref[...]
    h = jnp.maximum(z, 0.0)
    bids = b2d_ref[0, :]
    onehot = (bids[None, :]
              == lax.broadcasted_iota(jnp.int32, (NUM_GRAPHS, N_NODES), 0)
              ).astype(jnp.float32)
    pool_ref[...] = lax.dot_general(onehot, h, (((1,), (0,)), ((), ())),
                                    preferred_element_type=jnp.float32)


_mlp_last = pl.pallas_call(
    _mlp_last_body,
    out_shape=jax.ShapeDtypeStruct((NUM_GRAPHS, FEAT), jnp.float32),
)


def kernel(x, edge_index, batch,
           W1_0, b1_0, W2_0, b2_0,
           W1_1, b1_1, W2_1, b2_1,
           W1_2, b1_2, W2_2, b2_2):
    src = edge_index[0].reshape(NW, E_PER_W)
    dst = edge_index[1].reshape(NW, N_CHUNKS, CHUNK)
    batch2 = batch.reshape(1, N_NODES)
    params = [(W1_0, b1_0, W2_0, b2_0),
              (W1_1, b1_1, W2_1, b2_1),
              (W1_2, b1_2, W2_2, b2_2)]
    h = x.astype(jnp.float32)
    pools = []
    for li, (W1, b1, W2, b2) in enumerate(params):
        agg = _seg_sum(h, src, dst)
        args = (agg[0], agg[1], batch2,
                W1, b1.reshape(1, FEAT), W2, b2.reshape(1, FEAT))
        if li < len(params) - 1:
            h, pool = _mlp(h, *args)
        else:
            pool = _mlp_last(h, *args)
        pools.append(pool)
    return jnp.concatenate(pools, axis=1)


# speedup vs baseline: 1.0113x; 1.0113x over previous
import functools

import jax
import jax.numpy as jnp
from jax import lax
from jax.experimental import pallas as pl
from jax.experimental.pallas import tpu as pltpu
from jax.experimental.pallas import tpu_sc as plsc

N_NODES = 10000
N_EDGES = 320000
FEAT = 128
NUM_GRAPHS = 64

NC = 2
NS = 16
NW = NC * NS
E_PER_W = N_EDGES // NW
CHUNK = 80
N_CHUNKS = E_PER_W // CHUNK
ROWS_A = 624
ROWS_LAST = N_NODES - (NS - 1) * ROWS_A
ZR = 8

BLK = 1000
N_BLKS = N_NODES // BLK


def _seg_sum_body(h_hbm, src_hbm, dst_hbm, out_hbm,
                  src_v, dst_v, rows0_v, rows1_v, zbuf_v, agg_sh,
                  sem0, sem1, semz):
    c = lax.axis_index("c")
    s = lax.axis_index("s")
    wid = s * NC + c

    cs = pltpu.async_copy(src_hbm.at[wid], src_v, sem0)
    cd = pltpu.async_copy(dst_hbm.at[wid], dst_v, sem1)

    zeros16 = jnp.zeros((16,), jnp.float32)

    @pl.loop(0, ZR)
    def _zero_rows(r):
        for cc in range(FEAT // 16):
            zbuf_v[r, pl.ds(cc * 16, 16)] = zeros16

    r0 = s * ROWS_A

    nz = jnp.where(s == NS - 1, ROWS_LAST // ZR, ROWS_A // ZR)

    @pl.loop(0, nz)
    def _zero_issue(j):
        pltpu.async_copy(zbuf_v, agg_sh.at[pl.ds(r0 + j * ZR, ZR)], semz)

    cs.wait()
    cd.wait()
    pltpu.async_copy(h_hbm.at[src_v.at[pl.ds(0, CHUNK)]], rows0_v, sem0)
    pltpu.async_copy(h_hbm.at[src_v.at[pl.ds(CHUNK, CHUNK)]], rows1_v, sem1)

    @pl.loop(0, nz)
    def _zero_drain(j):
        pltpu.make_async_copy(zbuf_v, agg_sh.at[pl.ds(r0 + j * ZR, ZR)],
                              semz).wait()

    plsc.subcore_barrier()

    @pl.loop(0, N_CHUNKS - 1, step=2)
    def _edges(i):
        pltpu.make_async_copy(
            h_hbm.at[src_v.at[pl.ds(i * CHUNK, CHUNK)]], rows0_v, sem0).wait()
        pltpu.sync_copy(rows0_v, agg_sh.at[dst_v.at[i]], add=True)
        pltpu.async_copy(
            h_hbm.at[src_v.at[pl.ds((i + 2) * CHUNK, CHUNK)]], rows0_v, sem0)
        pltpu.make_async_copy(
            h_hbm.at[src_v.at[pl.ds((i + 1) * CHUNK, CHUNK)]], rows1_v, sem1).wait()
        pltpu.sync_copy(rows1_v, agg_sh.at[dst_v.at[i + 1]], add=True)

        @pl.when(i + 3 < N_CHUNKS)
        def _():
            pltpu.async_copy(
                h_hbm.at[src_v.at[pl.ds((i + 3) * CHUNK, CHUNK)]], rows1_v, sem1)

    pltpu.make_async_copy(
        h_hbm.at[src_v.at[pl.ds((N_CHUNKS - 1) * CHUNK, CHUNK)]],
        rows0_v, sem0).wait()
    pltpu.sync_copy(rows0_v, agg_sh.at[dst_v.at[N_CHUNKS - 1]], add=True)

    plsc.subcore_barrier()

    @pl.when(s < NS - 1)
    def _():
        pltpu.sync_copy(agg_sh.at[pl.ds(r0, ROWS_A)],
                        out_hbm.at[c, pl.ds(r0, ROWS_A)])

    @pl.when(s == NS - 1)
    def _():
        pltpu.sync_copy(agg_sh.at[pl.ds(r0, ROWS_LAST)],
                        out_hbm.at[c, pl.ds(r0, ROWS_LAST)])


_seg_sum = functools.partial(
    pl.kernel,
    out_type=jax.ShapeDtypeStruct((NC, N_NODES, FEAT), jnp.float32),
    mesh=plsc.VectorSubcoreMesh(core_axis_name="c", subcore_axis_name="s",
                                num_cores=NC, num_subcores=NS),
    scratch_types=[
        pltpu.VMEM((E_PER_W,), jnp.int32),
        pltpu.VMEM((N_CHUNKS, CHUNK), jnp.int32),
        pltpu.VMEM((CHUNK, FEAT), jnp.float32),
        pltpu.VMEM((CHUNK, FEAT), jnp.float32),
        pltpu.VMEM((ZR, FEAT), jnp.float32),
        pltpu.VMEM_SHARED((N_NODES, FEAT), jnp.float32),
        pltpu.SemaphoreType.DMA,
        pltpu.SemaphoreType.DMA,
        pltpu.SemaphoreType.DMA,
    ],
)(_seg_sum_body)


def _mlp_body(h_ref, a0_ref, a1_ref, b2d_ref, w1_ref, b1_ref, w2_ref, b2_ref,
              hout_ref, pool_ref):
    z = h_ref[...] + a0_ref[...] + a1_ref[...]
    z = jnp.maximum(
        jnp.dot(z, w1_ref[...], preferred_element_type=jnp.float32) + b1_ref[...],
        0.0)
    z = jnp.dot(z, w2_ref[...], preferred_element_type=jnp.float32) + b2_ref[...]
    h = jnp.maximum(z, 0.0)
    hout_ref[...] = h

    bids = b2d_ref[0, :]
    onehot = (bids[None, :]
              == lax.broadcasted_iota(jnp.int32, (NUM_GRAPHS, N_NODES), 0)
              ).astype(jnp.float32)
    pool_ref[...] = lax.dot_general(onehot, h, (((1,), (0,)), ((), ())),
                                    preferred_element_type=jnp.float32)


_mlp = pl.pallas_call(
    _mlp_body,
    out_shape=[
        jax.ShapeDtypeStruct((N_NODES, FEAT), jnp.float32),
        jax.ShapeDtypeStruct((NUM_GRAPHS, FEAT), jnp.float32),
    ],
)


def _mlp_last_body(h_ref, a0_ref, a1_ref, b2d_ref, w1_ref, b1_ref, w2_ref,
                   b2_ref, pool_ref):
    z = h_ref[...] + a0_ref[...] + a1_ref[...]
    z = jnp.maximum(
        jnp.dot(z, w1_ref[...], preferred_element_type=jnp.float32) + b1_ref[...],
        0.0)
    z = jnp.dot(z, w2_ref[...], preferred_element_type=jnp.float32) + b2_ref[...]
    h = jnp.maximum(z, 0.0)
    bids = b2d_ref[0, :]
    onehot = (bids[None, :]
              == lax.broadcasted_iota(jnp.int32, (NUM_GRAPHS, N_NODES), 0)
              ).astype(jnp.float32)
    pool_ref[...] = lax.dot_general(onehot, h, (((1,), (0,)), ((), ())),
                                    preferred_element_type=jnp.float32)


_mlp_last = pl.pallas_call(
    _mlp_last_body,
    out_shape=jax.ShapeDtypeStruct((NUM_GRAPHS, FEAT), jnp.float32),
)


def kernel(x, edge_index, batch,
           W1_0, b1_0, W2_0, b2_0,
           W1_1, b1_1, W2_1, b2_1,
           W1_2, b1_2, W2_2, b2_2):
    src = edge_index[0].reshape(NW, E_PER_W)
    dst = edge_index[1].reshape(NW, N_CHUNKS, CHUNK)
    batch2 = batch.reshape(1, N_NODES)
    params = [(W1_0, b1_0, W2_0, b2_0),
              (W1_1, b1_1, W2_1, b2_1),
              (W1_2, b1_2, W2_2, b2_2)]
    h = x.astype(jnp.float32)
    pools = []
    for li, (W1, b1, W2, b2) in enumerate(params):
        agg = _seg_sum(h, src, dst)
        args = (agg[0], agg[1], batch2,
                W1, b1.reshape(1, FEAT), W2, b2.reshape(1, FEAT))
        if li < len(params) - 1:
            h, pool = _mlp(h, *args)
        else:
            pool = _mlp_last(h, *args)
        pools.append(pool)
    return jnp.concatenate(pools, axis=1)
